# Initial kernel scaffold; baseline (speedup 1.0000x reference)
import jax
import jax.numpy as jnp
from jax.experimental import pallas as pl


def kernel(x, node_attr, edge_index, edge_attr, W_node_enc, b_node_enc, W_edge_enc, b_edge_enc, W_self, W_msg, b_proc, W_time, W_dec, b_dec):
    raise NotImplementedError("write your pallas kernel here")


# speedup vs baseline: 1.0297x; 1.0297x over previous
import jax
import jax.numpy as jnp
from jax import lax
from jax.experimental import pallas as pl
from jax.experimental.pallas import tpu as pltpu
from jax.experimental.pallas import tpu_sc as plsc

N_NODES = 50000
N_EDGES = 800000
T = 24
H = 32
DE = 16
ALPHA = 0.9

NC = 2
NS = 16
CHUNK = 128
NCHUNKS = 6400
EPAD = NCHUNKS * CHUNK
NPAD = 50208
NRANGES = 6
NQ = NPAD // NRANGES
GROWS = 8384
GDUMMY = NQ
ZROWS = GROWS // NS
CROWS = NQ // NS
TPC = T // NC
CPT_STATS = NCHUNKS // (NC * NS)
CPT = NCHUNKS // NS
KB = 8
NBATCH = CPT // KB
BN = 4184
BNE = 2048
BNF = 1000

f32 = jnp.float32


def _encode_body(attr_ref, w_ref, b_ref, out_ref):
    gl = attr_ref[:, 72:73]
    el = attr_ref[:, 73:74]
    base = gl * w_ref[3:4] + el * w_ref[4:5] + b_ref[...]
    for t in range(T):
        hx = attr_ref[:, t:t + 1]
        dh = attr_ref[:, T + t:T + t + 1]
        sl = attr_ref[:, 2 * T + t:2 * T + t + 1]
        out_ref[t] = hx * w_ref[0:1] + dh * w_ref[1:2] + sl * w_ref[2:3] + base


def _ce_body(s_ref, we_ref, wm_ref, be_ref, bp_ref, out_ref):
    a = jnp.dot(we_ref[...], wm_ref[...], preferred_element_type=f32)
    bv = jnp.dot(be_ref[...], wm_ref[...], preferred_element_type=f32)
    s = s_ref[0, :, 0:DE] + s_ref[1, :, 0:DE]
    cnt = s_ref[0, :, DE:DE + 1] + s_ref[1, :, DE:DE + 1]
    out_ref[...] = (jnp.dot(s, a, preferred_element_type=f32)
                    + cnt * bv + bp_ref[...])


def _dense_body(h_ref, g_ref, cf_ref, ws_ref, wm_ref, out_ref):
    h = h_ref[0]
    g = g_ref[0]
    out_ref[0] = jnp.maximum(
        jnp.dot(h, ws_ref[...], preferred_element_type=f32)
        + jnp.dot(g, wm_ref[...], preferred_element_type=f32)
        + cf_ref[...], 0.0)


def _final_body(sp_ref, x_ref, gl_ref, el_ref, wt_ref, wd_ref, bd_ref,
                out_ref):
    wd = wd_ref[...]
    cols = [jnp.sum(sp_ref[t] * wd, axis=1, keepdims=True) for t in range(T)]
    d1 = jnp.concatenate(cols, axis=1)
    tr = jnp.dot(d1, wt_ref[...], preferred_element_type=f32) + bd_ref[0, 0]
    el = el_ref[...]
    pred = ALPHA * tr + (1.0 - ALPHA) * (x_ref[...] - el) + el
    pred = jnp.minimum(pred, gl_ref[...])
    out_ref[...] = jnp.maximum(pred, el)


def _mk_idx(packed, j, sadj, deff, b, toff, qbase):
    for i in range(8):
        v = packed[j, pl.ds(i * 16, 16)]
        sadj[b, pl.ds(i * 16, 16)] = (v & 0xFFFF) + toff
        loc = lax.shift_right_logical(v, 16) - qbase
        ok = (loc >= 0) & (loc < NQ)
        deff[b, pl.ds(i * 16, 16)] = jnp.where(ok, loc, GDUMMY)


def _zero_acc(zeros_hbm, g_sh, s):
    pltpu.sync_copy(zeros_hbm, g_sh.at[pl.ds(s * ZROWS, ZROWS)])


def _prop_phase(hflat, packed, sadj, deff, rows, g_sh, gsem, zeros_hbm,
                g_out, s, toff, nr):
    def qloop(q, carry):
        _zero_acc(zeros_hbm, g_sh, s)
        plsc.subcore_barrier()
        qbase = q * NQ

        def batches(jj, carry2):
            j0 = jj * KB
            for b in range(KB):
                _mk_idx(packed, j0 + b, sadj, deff, b, toff, qbase)
            descs = [
                pltpu.async_copy(hflat.at[sadj.at[b]], rows.at[b], gsem)
                for b in range(KB)
            ]
            for b in range(KB):
                descs[b].wait()
                pltpu.sync_copy(rows.at[b], g_sh.at[deff.at[b]], add=True)
            return carry2

        lax.fori_loop(0, NBATCH, batches, 0)
        plsc.subcore_barrier()
        pltpu.sync_copy(g_sh.at[pl.ds(s * CROWS, CROWS)],
                        g_out.at[pl.ds(toff + qbase + s * CROWS, CROWS)])
        return carry

    lax.fori_loop(0, nr, qloop, 0)


def _stats_phase(ea32, packed, sadj, deff, rows, g_sh, s_out, zeros_hbm,
                 c, s, nr):
    wid = c * NS + s
    base = wid * CPT_STATS

    def qloop(q, carry):
        _zero_acc(zeros_hbm, g_sh, s)
        plsc.subcore_barrier()
        qbase = q * NQ

        def body(j, carry2):
            pltpu.sync_copy(ea32.at[base + j], rows.at[0])
            _mk_idx(packed, j, sadj, deff, 0, 0, qbase)
            pltpu.sync_copy(rows.at[0], g_sh.at[deff.at[0]], add=True)
            return carry2

        lax.fori_loop(0, CPT_STATS, body, 0)
        plsc.subcore_barrier()
        pltpu.sync_copy(g_sh.at[pl.ds(s * CROWS, CROWS)],
                        s_out.at[pl.ds(c * NPAD + qbase + s * CROWS, CROWS)])
        return carry

    lax.fori_loop(0, nr, qloop, 0)


def _prop_a_body(hflat, pk_hbm, ea32, zeros_hbm, cfg_hbm, g_out, s_out,
                 packed, sadj, deff, rows, cfgv, g_sh, gsem):
    c = lax.axis_index("c")
    s = lax.axis_index("s")
    pltpu.sync_copy(cfg_hbm, cfgv)
    nr = cfgv[pl.ds(0, 16)][0]
    wid = c * NS + s
    pltpu.sync_copy(pk_hbm.at[pl.ds(wid * CPT_STATS, CPT_STATS)],
                    packed.at[pl.ds(0, CPT_STATS)])
    _stats_phase(ea32, packed, sadj, deff, rows, g_sh, s_out, zeros_hbm,
                 c, s, nr)
    pltpu.sync_copy(pk_hbm.at[pl.ds(s * CPT, CPT)], packed)

    def tloop(tt, carry):
        t = c * TPC + tt
        _prop_phase(hflat, packed, sadj, deff, rows, g_sh, gsem, zeros_hbm,
                    g_out, s, t * NPAD, nr)
        return carry

    lax.fori_loop(0, TPC, tloop, 0)


def _prop_b_body(hflat, pk_hbm, zeros_hbm, cfg_hbm, g_out,
                 packed, sadj, deff, rows, cfgv, g_sh, gsem):
    c = lax.axis_index("c")
    s = lax.axis_index("s")
    pltpu.sync_copy(cfg_hbm, cfgv)
    nr = cfgv[pl.ds(0, 16)][0]
    pltpu.sync_copy(pk_hbm.at[pl.ds(s * CPT, CPT)], packed)

    def tloop(tt, carry):
        t = c * TPC + tt
        _prop_phase(hflat, packed, sadj, deff, rows, g_sh, gsem, zeros_hbm,
                    g_out, s, t * NPAD, nr)
        return carry

    lax.fori_loop(0, TPC, tloop, 0)


_SC_PARAMS = dict(
    mesh=plsc.VectorSubcoreMesh(core_axis_name="c", subcore_axis_name="s"),
    compiler_params=pltpu.CompilerParams(use_tc_tiling_on_sc=False),
)

_SC_SCRATCH = [
    pltpu.VMEM((CPT, CHUNK), jnp.int32),
    pltpu.VMEM((KB, CHUNK), jnp.int32),
    pltpu.VMEM((KB, CHUNK), jnp.int32),
    pltpu.VMEM((KB, CHUNK, H), f32),
    pltpu.VMEM((16,), jnp.int32),
    pltpu.VMEM_SHARED((GROWS, H), f32),
    pltpu.SemaphoreType.DMA,
]


def kernel(x, node_attr, edge_index, edge_attr, W_node_enc, b_node_enc,
           W_edge_enc, b_edge_enc, W_self, W_msg, b_proc, W_time, W_dec,
           b_dec):
    pad = EPAD - N_EDGES
    src_p = jnp.concatenate([edge_index[0], jnp.zeros((pad,), jnp.int32)])
    dst_p = jnp.concatenate(
        [edge_index[1], jnp.full((pad,), N_NODES, jnp.int32)])
    pk = (src_p | (dst_p << 16)).reshape(NCHUNKS, CHUNK)
    ea32 = jnp.concatenate(
        [edge_attr, jnp.ones((N_EDGES, 1), f32),
         jnp.zeros((N_EDGES, H - DE - 1), f32)], axis=1)
    ea32 = jnp.concatenate([ea32, jnp.zeros((pad, H), f32)], axis=0)
    ea32 = ea32.reshape(NCHUNKS, CHUNK, H)
    zeros32 = jnp.zeros((ZROWS, H), f32)
    cfg = jnp.full((16,), NRANGES, jnp.int32)

    coded = pl.pallas_call(
        _encode_body,
        grid=((NPAD + BNE - 1) // BNE,),
        in_specs=[
            pl.BlockSpec((BNE, 74), lambda j: (j, 0)),
            pl.BlockSpec((5, H), lambda j: (0, 0)),
            pl.BlockSpec((1, H), lambda j: (0, 0)),
        ],
        out_specs=pl.BlockSpec((T, BNE, H), lambda j: (0, j, 0)),
        out_shape=jax.ShapeDtypeStruct((T, NPAD, H), f32),
    )(node_attr, W_node_enc, b_node_enc.reshape(1, H))

    g1f, s32 = pl.kernel(
        _prop_a_body,
        out_type=(jax.ShapeDtypeStruct((T * NPAD, H), f32),
                  jax.ShapeDtypeStruct((NC * NPAD, H), f32)),
        scratch_types=_SC_SCRATCH,
        **_SC_PARAMS,
    )(coded.reshape(T * NPAD, H), pk, ea32, zeros32, cfg)

    cfull = pl.pallas_call(
        _ce_body,
        grid=(NPAD // BN,),
        in_specs=[
            pl.BlockSpec((NC, BN, H), lambda j: (0, j, 0)),
            pl.BlockSpec((DE, H), lambda j: (0, 0)),
            pl.BlockSpec((H, H), lambda j: (0, 0)),
            pl.BlockSpec((1, H), lambda j: (0, 0)),
            pl.BlockSpec((1, H), lambda j: (0, 0)),
        ],
        out_specs=pl.BlockSpec((BN, H), lambda j: (j, 0)),
        out_shape=jax.ShapeDtypeStruct((NPAD, H), f32),
    )(s32.reshape(NC, NPAD, H), W_edge_enc, W_msg,
      b_edge_enc.reshape(1, H), b_proc.reshape(1, H))

    prop_b = pl.kernel(
        _prop_b_body,
        out_type=jax.ShapeDtypeStruct((T * NPAD, H), f32),
        scratch_types=_SC_SCRATCH,
        **_SC_PARAMS,
    )

    def dense(h, g):
        return pl.pallas_call(
            _dense_body,
            grid=(T, NPAD // BN),
            in_specs=[
                pl.BlockSpec((1, BN, H), lambda i, j: (i, j, 0)),
                pl.BlockSpec((1, BN, H), lambda i, j: (i, j, 0)),
                pl.BlockSpec((BN, H), lambda i, j: (j, 0)),
                pl.BlockSpec((H, H), lambda i, j: (0, 0)),
                pl.BlockSpec((H, H), lambda i, j: (0, 0)),
            ],
            out_specs=pl.BlockSpec((1, BN, H), lambda i, j: (i, j, 0)),
            out_shape=jax.ShapeDtypeStruct((T, NPAD, H), f32),
        )(h, g, cfull, W_self, W_msg)

    h1 = dense(coded, g1f.reshape(T, NPAD, H))
    g2f = prop_b(h1.reshape(T * NPAD, H), pk, zeros32, cfg)
    h2 = dense(h1, g2f.reshape(T, NPAD, H))

    out = pl.pallas_call(
        _final_body,
        grid=(N_NODES // BNF,),
        in_specs=[
            pl.BlockSpec((T, BNF, H), lambda j: (0, j, 0)),
            pl.BlockSpec((BNF, T), lambda j: (j, 0)),
            pl.BlockSpec((BNF, 1), lambda j: (j, 0)),
            pl.BlockSpec((BNF, 1), lambda j: (j, 0)),
            pl.BlockSpec((T, T), lambda j: (0, 0)),
            pl.BlockSpec((1, H), lambda j: (0, 0)),
            pl.BlockSpec((1, 1), lambda j: (0, 0)),
        ],
        out_specs=pl.BlockSpec((BNF, T), lambda j: (j, 0)),
        out_shape=jax.ShapeDtypeStruct((N_NODES, T), f32),
    )(h2, x, node_attr[:, 72:73], node_attr[:, 73:74], W_time,
      W_dec.reshape(1, H), b_dec.reshape(1, 1))
    return out


# speedup vs baseline: 1.0302x; 1.0005x over previous
import jax
import jax.numpy as jnp
from jax import lax
from jax.experimental import pallas as pl
from jax.experimental.pallas import tpu as pltpu
from jax.experimental.pallas import tpu_sc as plsc

N_NODES = 50000
N_EDGES = 800000
T = 24
H = 32
DE = 16
ALPHA = 0.9

NC = 2
NS = 16
CHUNK = 128
NCHUNKS = 6400
EPAD = NCHUNKS * CHUNK
NPAD = 50208
NRANGES = 6
NQ = NPAD // NRANGES
GROWS = 8384
GDUMMY = NQ
ZROWS = GROWS // NS
CROWS = NQ // NS
TPC = T // NC
CPT_STATS = NCHUNKS // (NC * NS)
CPT = NCHUNKS // NS
KB = 10
NBATCH = CPT // KB
BN = 4184
BNE = 2048
BNF = 1000

f32 = jnp.float32


def _encode_body(attr_ref, w_ref, b_ref, out_ref):
    gl = attr_ref[:, 72:73]
    el = attr_ref[:, 73:74]
    base = gl * w_ref[3:4] + el * w_ref[4:5] + b_ref[...]
    for t in range(T):
        hx = attr_ref[:, t:t + 1]
        dh = attr_ref[:, T + t:T + t + 1]
        sl = attr_ref[:, 2 * T + t:2 * T + t + 1]
        out_ref[t] = hx * w_ref[0:1] + dh * w_ref[1:2] + sl * w_ref[2:3] + base


def _ce_body(s_ref, we_ref, wm_ref, be_ref, bp_ref, out_ref):
    a = jnp.dot(we_ref[...], wm_ref[...], preferred_element_type=f32)
    bv = jnp.dot(be_ref[...], wm_ref[...], preferred_element_type=f32)
    s = s_ref[0, :, 0:DE] + s_ref[1, :, 0:DE]
    cnt = s_ref[0, :, DE:DE + 1] + s_ref[1, :, DE:DE + 1]
    out_ref[...] = (jnp.dot(s, a, preferred_element_type=f32)
                    + cnt * bv + bp_ref[...])


def _dense_body(h_ref, g_ref, cf_ref, ws_ref, wm_ref, out_ref):
    h = h_ref[0]
    g = g_ref[0]
    out_ref[0] = jnp.maximum(
        jnp.dot(h, ws_ref[...], preferred_element_type=f32)
        + jnp.dot(g, wm_ref[...], preferred_element_type=f32)
        + cf_ref[...], 0.0)


def _final_body(sp_ref, x_ref, gl_ref, el_ref, wt_ref, wd_ref, bd_ref,
                out_ref):
    wd = wd_ref[...]
    cols = [jnp.sum(sp_ref[t] * wd, axis=1, keepdims=True) for t in range(T)]
    d1 = jnp.concatenate(cols, axis=1)
    tr = jnp.dot(d1, wt_ref[...], preferred_element_type=f32) + bd_ref[0, 0]
    el = el_ref[...]
    pred = ALPHA * tr + (1.0 - ALPHA) * (x_ref[...] - el) + el
    pred = jnp.minimum(pred, gl_ref[...])
    out_ref[...] = jnp.maximum(pred, el)


def _mk_idx(packed, j, sadj, deff, b, toff, qbase):
    for i in range(8):
        v = packed[j, pl.ds(i * 16, 16)]
        sadj[b, pl.ds(i * 16, 16)] = (v & 0xFFFF) + toff
        loc = lax.shift_right_logical(v, 16) - qbase
        ok = (loc >= 0) & (loc < NQ)
        deff[b, pl.ds(i * 16, 16)] = jnp.where(ok, loc, GDUMMY)


def _zero_acc(zeros_hbm, g_sh, s):
    pltpu.sync_copy(zeros_hbm, g_sh.at[pl.ds(s * ZROWS, ZROWS)])


def _prop_phase(hflat, packed, sadj, deff, rows, g_sh, gsem, zeros_hbm,
                g_out, s, toff, nr):
    def qloop(q, carry):
        _zero_acc(zeros_hbm, g_sh, s)
        plsc.subcore_barrier()
        qbase = q * NQ

        def batches(jj, carry2):
            j0 = jj * KB
            for b in range(KB):
                _mk_idx(packed, j0 + b, sadj, deff, b, toff, qbase)
            descs = [
                pltpu.async_copy(hflat.at[sadj.at[b]], rows.at[b], gsem)
                for b in range(KB)
            ]
            for b in range(KB):
                descs[b].wait()
                pltpu.sync_copy(rows.at[b], g_sh.at[deff.at[b]], add=True)
            return carry2

        lax.fori_loop(0, NBATCH, batches, 0)
        plsc.subcore_barrier()
        pltpu.sync_copy(g_sh.at[pl.ds(s * CROWS, CROWS)],
                        g_out.at[pl.ds(toff + qbase + s * CROWS, CROWS)])
        return carry

    lax.fori_loop(0, nr, qloop, 0)


def _stats_phase(ea32, packed, sadj, deff, rows, g_sh, s_out, zeros_hbm,
                 c, s, nr):
    wid = c * NS + s
    base = wid * CPT_STATS

    def qloop(q, carry):
        _zero_acc(zeros_hbm, g_sh, s)
        plsc.subcore_barrier()
        qbase = q * NQ

        def body(j, carry2):
            pltpu.sync_copy(ea32.at[base + j], rows.at[0])
            _mk_idx(packed, j, sadj, deff, 0, 0, qbase)
            pltpu.sync_copy(rows.at[0], g_sh.at[deff.at[0]], add=True)
            return carry2

        lax.fori_loop(0, CPT_STATS, body, 0)
        plsc.subcore_barrier()
        pltpu.sync_copy(g_sh.at[pl.ds(s * CROWS, CROWS)],
                        s_out.at[pl.ds(c * NPAD + qbase + s * CROWS, CROWS)])
        return carry

    lax.fori_loop(0, nr, qloop, 0)


def _prop_a_body(hflat, pk_hbm, ea32, zeros_hbm, cfg_hbm, g_out, s_out,
                 packed, sadj, deff, rows, cfgv, g_sh, gsem):
    c = lax.axis_index("c")
    s = lax.axis_index("s")
    pltpu.sync_copy(cfg_hbm, cfgv)
    nr = cfgv[pl.ds(0, 16)][0]
    wid = c * NS + s
    pltpu.sync_copy(pk_hbm.at[pl.ds(wid * CPT_STATS, CPT_STATS)],
                    packed.at[pl.ds(0, CPT_STATS)])
    _stats_phase(ea32, packed, sadj, deff, rows, g_sh, s_out, zeros_hbm,
                 c, s, nr)
    pltpu.sync_copy(pk_hbm.at[pl.ds(s * CPT, CPT)], packed)

    def tloop(tt, carry):
        t = c * TPC + tt
        _prop_phase(hflat, packed, sadj, deff, rows, g_sh, gsem, zeros_hbm,
                    g_out, s, t * NPAD, nr)
        return carry

    lax.fori_loop(0, TPC, tloop, 0)


def _prop_b_body(hflat, pk_hbm, zeros_hbm, cfg_hbm, g_out,
                 packed, sadj, deff, rows, cfgv, g_sh, gsem):
    c = lax.axis_index("c")
    s = lax.axis_index("s")
    pltpu.sync_copy(cfg_hbm, cfgv)
    nr = cfgv[pl.ds(0, 16)][0]
    pltpu.sync_copy(pk_hbm.at[pl.ds(s * CPT, CPT)], packed)

    def tloop(tt, carry):
        t = c * TPC + tt
        _prop_phase(hflat, packed, sadj, deff, rows, g_sh, gsem, zeros_hbm,
                    g_out, s, t * NPAD, nr)
        return carry

    lax.fori_loop(0, TPC, tloop, 0)


_SC_PARAMS = dict(
    mesh=plsc.VectorSubcoreMesh(core_axis_name="c", subcore_axis_name="s"),
    compiler_params=pltpu.CompilerParams(use_tc_tiling_on_sc=False),
)

_SC_SCRATCH = [
    pltpu.VMEM((CPT, CHUNK), jnp.int32),
    pltpu.VMEM((KB, CHUNK), jnp.int32),
    pltpu.VMEM((KB, CHUNK), jnp.int32),
    pltpu.VMEM((KB, CHUNK, H), f32),
    pltpu.VMEM((16,), jnp.int32),
    pltpu.VMEM_SHARED((GROWS, H), f32),
    pltpu.SemaphoreType.DMA,
]


def kernel(x, node_attr, edge_index, edge_attr, W_node_enc, b_node_enc,
           W_edge_enc, b_edge_enc, W_self, W_msg, b_proc, W_time, W_dec,
           b_dec):
    pad = EPAD - N_EDGES
    src_p = jnp.concatenate([edge_index[0], jnp.zeros((pad,), jnp.int32)])
    dst_p = jnp.concatenate(
        [edge_index[1], jnp.full((pad,), N_NODES, jnp.int32)])
    pk = (src_p | (dst_p << 16)).reshape(NCHUNKS, CHUNK)
    ea32 = jnp.concatenate(
        [edge_attr, jnp.ones((N_EDGES, 1), f32),
         jnp.zeros((N_EDGES, H - DE - 1), f32)], axis=1)
    ea32 = jnp.concatenate([ea32, jnp.zeros((pad, H), f32)], axis=0)
    ea32 = ea32.reshape(NCHUNKS, CHUNK, H)
    zeros32 = jnp.zeros((ZROWS, H), f32)
    cfg = jnp.full((16,), NRANGES, jnp.int32)

    coded = pl.pallas_call(
        _encode_body,
        grid=((NPAD + BNE - 1) // BNE,),
        in_specs=[
            pl.BlockSpec((BNE, 74), lambda j: (j, 0)),
            pl.BlockSpec((5, H), lambda j: (0, 0)),
            pl.BlockSpec((1, H), lambda j: (0, 0)),
        ],
        out_specs=pl.BlockSpec((T, BNE, H), lambda j: (0, j, 0)),
        out_shape=jax.ShapeDtypeStruct((T, NPAD, H), f32),
    )(node_attr, W_node_enc, b_node_enc.reshape(1, H))

    g1f, s32 = pl.kernel(
        _prop_a_body,
        out_type=(jax.ShapeDtypeStruct((T * NPAD, H), f32),
                  jax.ShapeDtypeStruct((NC * NPAD, H), f32)),
        scratch_types=_SC_SCRATCH,
        **_SC_PARAMS,
    )(coded.reshape(T * NPAD, H), pk, ea32, zeros32, cfg)

    cfull = pl.pallas_call(
        _ce_body,
        grid=(NPAD // BN,),
        in_specs=[
            pl.BlockSpec((NC, BN, H), lambda j: (0, j, 0)),
            pl.BlockSpec((DE, H), lambda j: (0, 0)),
            pl.BlockSpec((H, H), lambda j: (0, 0)),
            pl.BlockSpec((1, H), lambda j: (0, 0)),
            pl.BlockSpec((1, H), lambda j: (0, 0)),
        ],
        out_specs=pl.BlockSpec((BN, H), lambda j: (j, 0)),
        out_shape=jax.ShapeDtypeStruct((NPAD, H), f32),
    )(s32.reshape(NC, NPAD, H), W_edge_enc, W_msg,
      b_edge_enc.reshape(1, H), b_proc.reshape(1, H))

    prop_b = pl.kernel(
        _prop_b_body,
        out_type=jax.ShapeDtypeStruct((T * NPAD, H), f32),
        scratch_types=_SC_SCRATCH,
        **_SC_PARAMS,
    )

    def dense(h, g):
        return pl.pallas_call(
            _dense_body,
            grid=(T, NPAD // BN),
            in_specs=[
                pl.BlockSpec((1, BN, H), lambda i, j: (i, j, 0)),
                pl.BlockSpec((1, BN, H), lambda i, j: (i, j, 0)),
                pl.BlockSpec((BN, H), lambda i, j: (j, 0)),
                pl.BlockSpec((H, H), lambda i, j: (0, 0)),
                pl.BlockSpec((H, H), lambda i, j: (0, 0)),
            ],
            out_specs=pl.BlockSpec((1, BN, H), lambda i, j: (i, j, 0)),
            out_shape=jax.ShapeDtypeStruct((T, NPAD, H), f32),
        )(h, g, cfull, W_self, W_msg)

    h1 = dense(coded, g1f.reshape(T, NPAD, H))
    g2f = prop_b(h1.reshape(T * NPAD, H), pk, zeros32, cfg)
    h2 = dense(h1, g2f.reshape(T, NPAD, H))

    out = pl.pallas_call(
        _final_body,
        grid=(N_NODES // BNF,),
        in_specs=[
            pl.BlockSpec((T, BNF, H), lambda j: (0, j, 0)),
            pl.BlockSpec((BNF, T), lambda j: (j, 0)),
            pl.BlockSpec((BNF, 1), lambda j: (j, 0)),
            pl.BlockSpec((BNF, 1), lambda j: (j, 0)),
            pl.BlockSpec((T, T), lambda j: (0, 0)),
            pl.BlockSpec((1, H), lambda j: (0, 0)),
            pl.BlockSpec((1, 1), lambda j: (0, 0)),
        ],
        out_specs=pl.BlockSpec((BNF, T), lambda j: (j, 0)),
        out_shape=jax.ShapeDtypeStruct((N_NODES, T), f32),
    )(h2, x, node_attr[:, 72:73], node_attr[:, 73:74], W_time,
      W_dec.reshape(1, H), b_dec.reshape(1, 1))
    return out
